# baseline (speedup 1.0000x reference)
import functools

import jax
import jax.numpy as jnp
from jax import lax
from jax.experimental import pallas as pl
from jax.experimental.pallas import tpu as pltpu
from jax.experimental.pallas import tpu_sc as plsc

B = 16384
EMB = 64
SH = 32
NC = 2
NS = 16
NW = NC * NS
PER_W = B // NW
C = 128
NCH = PER_W // C
L = 16
NG = C // L

_DN = lax.GatherDimensionNumbers(
    offset_dims=(), collapsed_slice_dims=(0,), start_index_map=(0,))


def _lane_swap(v, perm2d):
    return lax.gather(v, perm2d, _DN, slice_sizes=(1,),
                      mode=lax.GatherScatterMode.PROMISE_IN_BOUNDS)


@functools.partial(
    pl.kernel,
    mesh=plsc.VectorSubcoreMesh(core_axis_name="c", subcore_axis_name="s"),
    out_type=jax.ShapeDtypeStruct((B,), jnp.float32),
    compiler_params=pltpu.CompilerParams(use_tc_tiling_on_sc=False),
    scratch_types=[
        pltpu.VMEM((C,), jnp.int32),
        pltpu.VMEM((C,), jnp.int32),
        pltpu.VMEM((C, EMB), jnp.float32),
        pltpu.VMEM((C, EMB), jnp.float32),
        pltpu.VMEM((C, SH), jnp.float32),
        pltpu.VMEM((C, SH), jnp.float32),
        pltpu.VMEM((C, SH), jnp.float32),
        pltpu.VMEM((C, SH), jnp.float32),
        pltpu.VMEM((C,), jnp.float32),
        pltpu.VMEM((C,), jnp.float32),
        pltpu.VMEM((C,), jnp.float32),
        pltpu.SemaphoreType.DMA,
    ],
)
def _shadow_mf(u_id_hbm, i_id_hbm, ush_hbm, ish_hbm,
               ue_hbm, bu_hbm, ie_hbm, bi_hbm, su_hbm, si_hbm,
               out_hbm,
               uidx_v, iidx_v, ue_v, ie_v, si_v, su_v, ush_v, ish_v,
               bu_v, bi_v, out_v, sem):
    wid = lax.axis_index("s") * NC + lax.axis_index("c")
    lane = lax.iota(jnp.int32, 16)
    masks = [(lane & s) == 0 for s in (1, 2, 4, 8)]
    perms = [(lane ^ s).reshape(16, 1) for s in (1, 2, 4, 8)]

    for ch in range(NCH):
        base = wid * PER_W + ch * C
        pltpu.sync_copy(u_id_hbm.at[pl.ds(base, C)], uidx_v)
        pltpu.sync_copy(i_id_hbm.at[pl.ds(base, C)], iidx_v)
        cps = [
            pltpu.async_copy(ue_hbm.at[uidx_v], ue_v, sem),
            pltpu.async_copy(ie_hbm.at[iidx_v], ie_v, sem),
            pltpu.async_copy(si_hbm.at[iidx_v], si_v, sem),
            pltpu.async_copy(su_hbm.at[uidx_v], su_v, sem),
            pltpu.async_copy(bu_hbm.at[uidx_v], bu_v, sem),
            pltpu.async_copy(bi_hbm.at[iidx_v], bi_v, sem),
        ]
        pltpu.sync_copy(ush_hbm.at[pl.ds(base, C)], ush_v)
        pltpu.sync_copy(ish_hbm.at[pl.ds(base, C)], ish_v)
        for cp in cps:
            cp.wait()

        def group(g, carry):
            vecs = []
            for e in range(L):
                r = g * L + e
                acc0 = ue_v[r, pl.ds(0, 16)] * ie_v[r, pl.ds(0, 16)]
                acc1 = ue_v[r, pl.ds(16, 16)] * ie_v[r, pl.ds(16, 16)]
                acc2 = ue_v[r, pl.ds(32, 16)] * ie_v[r, pl.ds(32, 16)]
                acc3 = ue_v[r, pl.ds(48, 16)] * ie_v[r, pl.ds(48, 16)]
                acc0 += ush_v[r, pl.ds(0, 16)] * si_v[r, pl.ds(0, 16)]
                acc1 += ush_v[r, pl.ds(16, 16)] * si_v[r, pl.ds(16, 16)]
                acc2 += ish_v[r, pl.ds(0, 16)] * su_v[r, pl.ds(0, 16)]
                acc3 += ish_v[r, pl.ds(16, 16)] * su_v[r, pl.ds(16, 16)]
                vecs.append((acc0 + acc1) + (acc2 + acc3))
            for lv, (m, p) in enumerate(zip(masks, perms)):
                nxt = []
                for j in range(0, len(vecs), 2):
                    a, b = vecs[j], vecs[j + 1]
                    x = jnp.where(m, a, b)
                    y = jnp.where(m, b, a)
                    nxt.append(x + _lane_swap(y, p))
                vecs = nxt
            res = vecs[0] + bu_v[pl.ds(g * L, L)] + bi_v[pl.ds(g * L, L)]
            out_v[pl.ds(g * L, L)] = res
            return carry

        lax.fori_loop(0, NG, group, 0)
        pltpu.sync_copy(out_v, out_hbm.at[pl.ds(base, C)])


def kernel(u_id, i_id, UserShadow, ItemShadow, user_emb_w, user_bias_w,
           item_emb_w, item_bias_w, shadow_u_w, shadow_i_w, mean):
    out = _shadow_mf(u_id.astype(jnp.int32), i_id.astype(jnp.int32),
                     UserShadow, ItemShadow,
                     user_emb_w, user_bias_w.reshape(-1),
                     item_emb_w, item_bias_w.reshape(-1),
                     shadow_u_w, shadow_i_w)
    return out + mean[0]
